# Initial kernel scaffold; baseline (speedup 1.0000x reference)
import functools

import jax
import jax.numpy as jnp
from jax import lax
from jax.experimental import pallas as pl
from jax.experimental.pallas import tpu as pltpu
from jax.experimental.pallas import tpu_sc as plsc

N = 10000
D = 128
E = 320000
MIN_NORM = 1e-15
PROJ_EPS = 4e-3

NC = 2
NS = 16
NW = NC * NS
L = 16

B = 128
EPT = 10240
EP = NW * EPT
NCH = EPT // B
RPS = N // NS


def _logmap_body(x_ref, o_ref):
    x = x_ref[...]
    nrm = jnp.maximum(jnp.sqrt(jnp.sum(x * x, axis=1, keepdims=True)), MIN_NORM)
    t = jnp.clip(nrm, -1.0 + 1e-5, 1.0 - 1e-5)
    at = 0.5 * jnp.log((1.0 + t) / (1.0 - t))
    o_ref[...] = x * (at / nrm)


def _logmap(x):
    return pl.pallas_call(
        _logmap_body,
        grid=(8,),
        in_specs=[pl.BlockSpec((N // 8, D), lambda i: (i, 0))],
        out_specs=pl.BlockSpec((N // 8, D), lambda i: (i, 0)),
        out_shape=jax.ShapeDtypeStruct((N, D), jnp.float32),
    )(x)


def _spmm_body(xt_hbm, row_hbm, col_hbm, val_hbm, zero_hbm, out_hbm,
               rowv, colv, valv, msgs, acc, sem):
    cid = lax.axis_index("c")
    sid = lax.axis_index("s")
    wid = cid * NS + sid

    pltpu.sync_copy(zero_hbm, acc.at[pl.ds(sid * RPS, RPS)])
    plsc.subcore_barrier()

    ebase = wid * EPT

    def chunk(ci, carry):
        base = ebase + ci * B
        pltpu.sync_copy(row_hbm.at[pl.ds(base, B)], rowv.at[0])
        pltpu.sync_copy(col_hbm.at[pl.ds(base, B)], colv)
        pltpu.sync_copy(val_hbm.at[pl.ds(base, B)], valv)
        pltpu.async_copy(xt_hbm.at[colv], msgs, sem).wait()

        def srow(b, c2):
            v = plsc.load_gather(valv, [jnp.full((L,), b, jnp.int32)])
            for k in range(D // L):
                msgs[b, pl.ds(k * L, L)] = msgs[b, pl.ds(k * L, L)] * v
            return c2

        lax.fori_loop(0, B, srow, 0)
        pltpu.sync_copy(msgs, acc.at[rowv.at[0]], add=True)
        return carry

    lax.fori_loop(0, NCH, chunk, 0)
    plsc.subcore_barrier()

    pltpu.sync_copy(acc.at[pl.ds(sid * RPS, RPS)],
                    out_hbm.at[pl.ds(cid * N + sid * RPS, RPS)])


def _spmm(xt, rows, cols, vals, zeros):
    mesh = plsc.VectorSubcoreMesh(core_axis_name="c", subcore_axis_name="s")
    f = functools.partial(
        pl.kernel,
        mesh=mesh,
        out_type=jax.ShapeDtypeStruct((2 * N, D), jnp.float32),
        scratch_types=[
            pltpu.VMEM((1, B), jnp.int32),
            pltpu.VMEM((B,), jnp.int32),
            pltpu.VMEM((B,), jnp.float32),
            pltpu.VMEM((B, D), jnp.float32),
            pltpu.VMEM_SHARED((N, D), jnp.float32),
            pltpu.SemaphoreType.DMA,
        ],
    )(_spmm_body)
    return f(xt, rows, cols, vals, zeros)


def _post_body(a_ref, b_ref, o_ref):
    u = a_ref[...] + b_ref[...]
    un = jnp.maximum(jnp.sqrt(jnp.sum(u * u, axis=1, keepdims=True)), MIN_NORM)
    y = jnp.tanh(un) * (u / un)
    yn = jnp.maximum(jnp.sqrt(jnp.sum(y * y, axis=1, keepdims=True)), MIN_NORM)
    maxnorm = 1.0 - PROJ_EPS
    o_ref[...] = jnp.where(yn > maxnorm, y / yn * maxnorm, y)


def _post(a, b):
    return pl.pallas_call(
        _post_body,
        grid=(8,),
        in_specs=[pl.BlockSpec((N // 8, D), lambda i: (i, 0)),
                  pl.BlockSpec((N // 8, D), lambda i: (i, 0))],
        out_specs=pl.BlockSpec((N // 8, D), lambda i: (i, 0)),
        out_shape=jax.ShapeDtypeStruct((N, D), jnp.float32),
    )(a, b)


def kernel(x, adj_indices, adj_values):
    xt = _logmap(x)
    pad = EP - E
    rows = jnp.pad(adj_indices[0], (0, pad))
    cols = jnp.pad(adj_indices[1], (0, pad))
    vals = jnp.pad(adj_values, (0, pad))
    zeros = jnp.zeros((RPS, D), jnp.float32)
    parts = _spmm(xt, rows, cols, vals, zeros)
    return _post(parts[:N], parts[N:])


# speedup vs baseline: 2.2003x; 2.2003x over previous
import functools

import jax
import jax.numpy as jnp
from jax import lax
from jax.experimental import pallas as pl
from jax.experimental.pallas import tpu as pltpu
from jax.experimental.pallas import tpu_sc as plsc

N = 10000
D = 128
E = 320000
MIN_NORM = 1e-15
PROJ_EPS = 4e-3

NC = 2
NS = 16
NW = NC * NS
L = 16

B = 128
EPT = 10240
EP = NW * EPT
NCH = EPT // B
NP = 10240
RPS = NP // NS


def _logmap_body(x_ref, o_ref):
    x = x_ref[...]
    nrm = jnp.maximum(jnp.sqrt(jnp.sum(x * x, axis=1, keepdims=True)), MIN_NORM)
    t = jnp.clip(nrm, -1.0 + 1e-5, 1.0 - 1e-5)
    at = 0.5 * jnp.log((1.0 + t) / (1.0 - t))
    o_ref[...] = x * (at / nrm)


def _logmap(x):
    return pl.pallas_call(
        _logmap_body,
        grid=(10,),
        in_specs=[pl.BlockSpec((N // 10, D), lambda i: (i, 0))],
        out_specs=pl.BlockSpec((N // 10, D), lambda i: (i, 0)),
        out_shape=jax.ShapeDtypeStruct((N, D), jnp.float32),
    )(x)


def _spmm_body(xt_hbm, row_hbm, col_hbm, val_hbm, zero_hbm, out_hbm,
               rowv, colv, valv, msgs, acc, sem):
    cid = lax.axis_index("c")
    sid = lax.axis_index("s")
    wid = cid * NS + sid

    pltpu.sync_copy(zero_hbm, acc.at[pl.ds(sid * RPS, RPS)])
    plsc.subcore_barrier()

    ebase = wid * EPT

    def chunk(ci, carry):
        base = ebase + ci * B
        pltpu.sync_copy(row_hbm.at[pl.ds(base, B)], rowv.at[0])
        pltpu.sync_copy(col_hbm.at[pl.ds(base, B)], colv)
        pltpu.sync_copy(val_hbm.at[pl.ds(base, B)], valv)
        pltpu.async_copy(xt_hbm.at[colv], msgs, sem).wait()

        def srow(b, c2):
            v = valv[b, :]
            for k in range(D // L):
                msgs[b, pl.ds(k * L, L)] = msgs[b, pl.ds(k * L, L)] * v
            return c2

        lax.fori_loop(0, B, srow, 0)
        pltpu.sync_copy(msgs, acc.at[rowv.at[0]], add=True)
        return carry

    lax.fori_loop(0, NCH, chunk, 0)
    plsc.subcore_barrier()

    pltpu.sync_copy(acc.at[pl.ds(sid * RPS, RPS)],
                    out_hbm.at[pl.ds(cid * NP + sid * RPS, RPS)])


def _spmm(xt, rows, cols, vals, zeros):
    mesh = plsc.VectorSubcoreMesh(core_axis_name="c", subcore_axis_name="s")
    f = functools.partial(
        pl.kernel,
        mesh=mesh,
        out_type=jax.ShapeDtypeStruct((2 * NP, D), jnp.float32),
        scratch_types=[
            pltpu.VMEM((1, B), jnp.int32),
            pltpu.VMEM((B,), jnp.int32),
            pltpu.VMEM((B, L), jnp.float32),
            pltpu.VMEM((B, D), jnp.float32),
            pltpu.VMEM_SHARED((NP, D), jnp.float32),
            pltpu.SemaphoreType.DMA,
        ],
    )(_spmm_body)
    return f(xt, rows, cols, vals, zeros)


def _post_body(a_ref, b_ref, o_ref):
    u = a_ref[...] + b_ref[...]
    un = jnp.maximum(jnp.sqrt(jnp.sum(u * u, axis=1, keepdims=True)), MIN_NORM)
    y = jnp.tanh(un) * (u / un)
    yn = jnp.maximum(jnp.sqrt(jnp.sum(y * y, axis=1, keepdims=True)), MIN_NORM)
    maxnorm = 1.0 - PROJ_EPS
    o_ref[...] = jnp.where(yn > maxnorm, y / yn * maxnorm, y)


def _post(a, b):
    return pl.pallas_call(
        _post_body,
        grid=(10,),
        in_specs=[pl.BlockSpec((N // 10, D), lambda i: (i, 0)),
                  pl.BlockSpec((N // 10, D), lambda i: (i, 0))],
        out_specs=pl.BlockSpec((N // 10, D), lambda i: (i, 0)),
        out_shape=jax.ShapeDtypeStruct((N, D), jnp.float32),
    )(a, b)


def kernel(x, adj_indices, adj_values):
    xt = _logmap(x)
    pad = EP - E
    rows = jnp.pad(adj_indices[0], (0, pad))
    cols = jnp.pad(adj_indices[1], (0, pad))
    vals = jnp.pad(adj_values, (0, pad))
    vals = jnp.broadcast_to(vals[:, None], (EP, L))
    zeros = jnp.zeros((RPS, D), jnp.float32)
    parts = _spmm(xt, rows, cols, vals, zeros)
    return _post(parts[:N], parts[NP:NP + N])


# speedup vs baseline: 3.3540x; 1.5244x over previous
import functools

import jax
import jax.numpy as jnp
from jax import lax
from jax.experimental import pallas as pl
from jax.experimental.pallas import tpu as pltpu
from jax.experimental.pallas import tpu_sc as plsc

N = 10000
D = 128
E = 320000
MIN_NORM = 1e-15
PROJ_EPS = 4e-3

NC = 2
NS = 16
NW = NC * NS
L = 16

B = 80
EPT = 10240
EP = NW * EPT
NCH = EPT // B
NP = 10240
RPS = NP // NS


def _logmap_body(x_ref, o_ref):
    x = x_ref[...]
    nrm = jnp.maximum(jnp.sqrt(jnp.sum(x * x, axis=1, keepdims=True)), MIN_NORM)
    t = jnp.clip(nrm, -1.0 + 1e-5, 1.0 - 1e-5)
    at = 0.5 * jnp.log((1.0 + t) / (1.0 - t))
    o_ref[...] = x * (at / nrm)


def _logmap(x):
    return pl.pallas_call(
        _logmap_body,
        grid=(10,),
        in_specs=[pl.BlockSpec((N // 10, D), lambda i: (i, 0))],
        out_specs=pl.BlockSpec((N // 10, D), lambda i: (i, 0)),
        out_shape=jax.ShapeDtypeStruct((N, D), jnp.float32),
    )(x)


NBUF = 4
UNROLL = 4


def _spmm_body(xt_hbm, row_hbm, col_hbm, val_hbm, zero_hbm, out_hbm,
               rowv, colv, valv, msgs, acc, *sems):
    rsems = sems[0:NBUF]
    csems = sems[NBUF:2 * NBUF]
    vsems = sems[2 * NBUF:3 * NBUF]
    gsems = sems[3 * NBUF:4 * NBUF]
    ssems = sems[4 * NBUF:5 * NBUF]
    cid = lax.axis_index("c")
    sid = lax.axis_index("s")
    wid = cid * NS + sid

    pltpu.sync_copy(zero_hbm, acc.at[pl.ds(sid * RPS, RPS)])
    plsc.subcore_barrier()

    def idx_start(ci, s):
        pltpu.async_copy(row_hbm.at[wid, ci], rowv.at[s], rsems[s])
        pltpu.async_copy(col_hbm.at[wid, ci], colv.at[s], csems[s])
        pltpu.async_copy(val_hbm.at[wid, ci], valv.at[s], vsems[s])

    def col_wait(ci, s):
        pltpu.make_async_copy(col_hbm.at[wid, ci], colv.at[s],
                              csems[s]).wait()

    def rowval_wait(ci, s):
        pltpu.make_async_copy(row_hbm.at[wid, ci], rowv.at[s],
                              rsems[s]).wait()
        pltpu.make_async_copy(val_hbm.at[wid, ci], valv.at[s],
                              vsems[s]).wait()

    def gather_start(ci, s):
        pltpu.async_copy(xt_hbm.at[colv.at[s]], msgs.at[s], gsems[s])

    def gather_wait(ci, s):
        pltpu.make_async_copy(xt_hbm.at[colv.at[s]], msgs.at[s],
                              gsems[s]).wait()

    def scatter_start(ci, s):
        pltpu.async_copy(msgs.at[s], acc.at[rowv.at[s]], ssems[s], add=True)

    def scatter_wait(ci, s):
        pltpu.make_async_copy(msgs.at[s], acc.at[rowv.at[s]],
                              ssems[s]).wait()

    idx_start(0, 0)
    idx_start(1, 1)
    col_wait(0, 0)
    gather_start(0, 0)

    def outer(g, carry):
        for s in range(NBUF):
            ci = g * NBUF + s
            gather_wait(ci, s)
            rowval_wait(ci, s)

            @pl.when(ci >= 2)
            def _():
                scatter_wait(ci - 2, (s - 2) % NBUF)

            @pl.when(ci + 2 < NCH)
            def _():
                idx_start(ci + 2, (s + 2) % NBUF)

            @pl.when(ci + 1 < NCH)
            def _():
                col_wait(ci + 1, (s + 1) % NBUF)
                gather_start(ci + 1, (s + 1) % NBUF)

            def srow(r, c2):
                for u in range(UNROLL):
                    b0 = r * UNROLL + u
                    v = valv[s, pl.ds(b0 * L, L)]
                    for k in range(D // L):
                        msgs[s, b0, pl.ds(k * L, L)] = (
                            msgs[s, b0, pl.ds(k * L, L)] * v)
                return c2

            lax.fori_loop(0, B // UNROLL, srow, 0)
            scatter_start(ci, s)
        return carry

    lax.fori_loop(0, NCH // NBUF, outer, 0)
    scatter_wait(NCH - 2, (NCH - 2) % NBUF)
    scatter_wait(NCH - 1, (NCH - 1) % NBUF)
    plsc.subcore_barrier()

    pltpu.sync_copy(acc.at[pl.ds(sid * RPS, RPS)],
                    out_hbm.at[pl.ds(cid * NP + sid * RPS, RPS)])


def _spmm(xt, rows, cols, vals, zeros):
    mesh = plsc.VectorSubcoreMesh(core_axis_name="c", subcore_axis_name="s")
    f = functools.partial(
        pl.kernel,
        mesh=mesh,
        out_type=jax.ShapeDtypeStruct((2 * NP, D), jnp.float32),
        scratch_types=[
            pltpu.VMEM((NBUF, B), jnp.int32),
            pltpu.VMEM((NBUF, B), jnp.int32),
            pltpu.VMEM((NBUF, B * L), jnp.float32),
            pltpu.VMEM((NBUF, B, D), jnp.float32),
            pltpu.VMEM_SHARED((NP, D), jnp.float32),
        ] + [pltpu.SemaphoreType.DMA] * (5 * NBUF),
    )(_spmm_body)
    return f(xt, rows, cols, vals, zeros)


def _post_body(a_ref, b_ref, o_ref):
    u = a_ref[...] + b_ref[...]
    un = jnp.maximum(jnp.sqrt(jnp.sum(u * u, axis=1, keepdims=True)), MIN_NORM)
    y = jnp.tanh(un) * (u / un)
    yn = jnp.maximum(jnp.sqrt(jnp.sum(y * y, axis=1, keepdims=True)), MIN_NORM)
    maxnorm = 1.0 - PROJ_EPS
    o_ref[...] = jnp.where(yn > maxnorm, y / yn * maxnorm, y)


def _post(a, b):
    return pl.pallas_call(
        _post_body,
        grid=(10,),
        in_specs=[pl.BlockSpec((N // 10, D), lambda i: (i, 0)),
                  pl.BlockSpec((N // 10, D), lambda i: (i, 0))],
        out_specs=pl.BlockSpec((N // 10, D), lambda i: (i, 0)),
        out_shape=jax.ShapeDtypeStruct((N, D), jnp.float32),
    )(a, b)


def kernel(x, adj_indices, adj_values):
    xt = _logmap(x)
    pad = EP - E
    rows = jnp.pad(adj_indices[0], (0, pad)).reshape(NW, NCH, B)
    cols = jnp.pad(adj_indices[1], (0, pad)).reshape(NW, NCH, B)
    vals = jnp.pad(adj_values, (0, pad))
    vals = jnp.broadcast_to(vals[:, None], (EP, L)).reshape(NW, NCH, B * L)
    zeros = jnp.zeros((RPS, D), jnp.float32)
    parts = _spmm(xt, rows, cols, vals, zeros)
    return _post(parts[:N], parts[NP:NP + N])
